# baseline (speedup 1.0000x reference)
import functools

import jax
import jax.numpy as jnp
from jax import lax
from jax.experimental import pallas as pl
from jax.experimental.pallas import tpu as pltpu
from jax.experimental.pallas import tpu_sc as plsc

N = 10000
NP = 10240
E = 320000
IN_CH = 128
HID = 256
OUT_CH = 128
TECS = 16
K = 128
EPT = E // TECS
CH = 160
PAD = CH * K - EPT
G = 32
NG = CH // G
EPT2 = E // 32
KE = 96
CHE = 210
PADE = CHE * KE - EPT
GE = 42
NGE = CHE // GE
CHE2 = 105
PADE2 = CHE2 * KE - EPT2
GE2 = 21
NGE2 = CHE2 // GE2
RP = NP // TECS

def _mesh():
    return plsc.VectorSubcoreMesh(core_axis_name="c", subcore_axis_name="s",
                                  num_cores=2, num_subcores=TECS)


def _deg_body(dst_hbm, out_hbm, dst_v, ones_v, acc):
    c = lax.axis_index("c")
    s = lax.axis_index("s")

    def fill(i, carry):
        ones_v[pl.ds(i * 16, 16)] = jnp.full((16,), 1.0, jnp.float32)
        return carry

    lax.fori_loop(0, NP // 16, fill, 0)
    pltpu.sync_copy(dst_hbm.at[s], dst_v)

    @pl.when(s == 0)
    def _():
        pltpu.sync_copy(ones_v, acc)

    plsc.subcore_barrier()

    half = CH // 2
    def body(j, carry):
        pltpu.sync_copy(ones_v.at[pl.ds(0, K)], acc.at[dst_v.at[j]], add=True)
        return carry

    lax.fori_loop(half * c, half * (c + 1), body, 0)
    plsc.subcore_barrier()
    pltpu.sync_copy(acc.at[pl.ds(s * RP, RP)], out_hbm.at[c, pl.ds(s * RP, RP)])


def _make_edge_body(colsplit, g_, ng_):
    def body_fn(ys_hbm, src_hbm, dst_hbm, out_hbm,
                src_v, dst_v, rows0, rows1, rows2, acc, sem0, sem1, sem2):
        c = lax.axis_index("c")
        s = lax.axis_index("s")
        rows = (rows0, rows1, rows2)
        sems = (sem0, sem1, sem2)
        init = (c * NP if colsplit else 0) + s * RP
        pltpu.sync_copy(ys_hbm.at[pl.ds(init, RP)], acc.at[pl.ds(s * RP, RP)])
        plsc.subcore_barrier()

        def group(g, carry):
            pltpu.sync_copy(src_hbm.at[c, s, g], src_v)
            pltpu.sync_copy(dst_hbm.at[c, s, g], dst_v)
            pltpu.async_copy(ys_hbm.at[src_v.at[0]], rows0, sem0)
            pltpu.async_copy(ys_hbm.at[src_v.at[1]], rows1, sem1)

            def tri(i, carry2):
                j = 3 * i
                for t in range(3):
                    b, bn = rows[t], rows[(t + 2) % 3]
                    sm, smn = sems[t], sems[(t + 2) % 3]
                    pltpu.make_async_copy(
                        ys_hbm.at[src_v.at[j + t]], b, sm).wait()

                    @pl.when(j + t + 2 < g_)
                    def _():
                        pltpu.async_copy(
                            ys_hbm.at[src_v.at[j + t + 2]], bn, smn)

                    pltpu.sync_copy(b, acc.at[dst_v.at[j + t]], add=True)
                return carry2

            return lax.fori_loop(0, g_ // 3, tri, carry)

        lax.fori_loop(0, ng_, group, 0)
        plsc.subcore_barrier()
        pltpu.sync_copy(acc.at[pl.ds(s * RP, RP)],
                        out_hbm.at[pl.ds(c * NP + s * RP, RP)])

    return body_fn


@functools.cache
def _sc_kernels():
    deg = pl.kernel(
        _deg_body,
        out_type=jax.ShapeDtypeStruct((2, NP), jnp.float32),
        mesh=_mesh(),
        scratch_types=[
            pltpu.VMEM((CH, K), jnp.int32),
            pltpu.VMEM((NP,), jnp.float32),
            pltpu.VMEM_SHARED((NP,), jnp.float32),
        ],
    )
    def edge(colsplit, g, ng, dt):
        return pl.kernel(
            _make_edge_body(colsplit, g, ng),
            out_type=jax.ShapeDtypeStruct((2 * NP, 128), dt),
            mesh=_mesh(),
            scratch_types=[
                pltpu.VMEM((g, KE), jnp.int32),
                pltpu.VMEM((g, KE), jnp.int32),
                pltpu.VMEM((KE, 128), dt),
                pltpu.VMEM((KE, 128), dt),
                pltpu.VMEM((KE, 128), dt),
                pltpu.VMEM_SHARED((NP, 128), dt),
                pltpu.SemaphoreType.DMA,
                pltpu.SemaphoreType.DMA,
                pltpu.SemaphoreType.DMA,
            ],
        )
    return (deg, edge(True, GE, NGE, jnp.float32),
            edge(False, GE2, NGE2, jnp.float32))


_TILE = 400
_GRID = N // _TILE


def _dinv_of(degp_ref):
    d = degp_ref[0, :, 0] + degp_ref[1, :, 0] - 1.0
    return lax.rsqrt(jnp.maximum(d, 1.0))


def _mm1_body(x_ref, w_ref, degp_ref, o_ref):
    dinv = _dinv_of(degp_ref)
    y = jnp.dot(x_ref[...], w_ref[...],
                preferred_element_type=jnp.float32) * dinv[:, None]
    o_ref[0] = y[:, :128]
    o_ref[1] = y[:, 128:]


def _mm1(xp, W1, degp):
    return pl.pallas_call(
        _mm1_body,
        grid=(_GRID,),
        in_specs=[
            pl.BlockSpec((_TILE, IN_CH), lambda i: (i, 0)),
            pl.BlockSpec((IN_CH, HID), lambda i: (0, 0)),
            pl.BlockSpec((2, _TILE, 1), lambda i: (0, i, 0)),
        ],
        out_specs=pl.BlockSpec((2, _TILE, 128), lambda i: (0, i, 0)),
        out_shape=jax.ShapeDtypeStruct((2, NP, 128), jnp.float32),
    )(xp, W1, degp)


def _mm2_body(acc_ref, degp_ref, b1_ref, w_ref, o_ref):
    dinv = _dinv_of(degp_ref)
    a = jnp.concatenate([acc_ref[0], acc_ref[1]], axis=1)
    h = jax.nn.relu(a * dinv[:, None] + b1_ref[0][None, :])
    y2 = jnp.dot(h, w_ref[...],
                 preferred_element_type=jnp.float32) * dinv[:, None]
    o_ref[...] = y2


def _mm2(acc1, degp, b1, W2):
    return pl.pallas_call(
        _mm2_body,
        grid=(_GRID,),
        in_specs=[
            pl.BlockSpec((2, _TILE, 128), lambda i: (0, i, 0)),
            pl.BlockSpec((2, _TILE, 1), lambda i: (0, i, 0)),
            pl.BlockSpec((1, HID), lambda i: (0, 0)),
            pl.BlockSpec((HID, OUT_CH), lambda i: (0, 0)),
        ],
        out_specs=pl.BlockSpec((_TILE, OUT_CH), lambda i: (i, 0)),
        out_shape=jax.ShapeDtypeStruct((NP, OUT_CH), jnp.float32),
    )(acc1, degp, b1, W2)


def _mm3_body(acc_ref, y2_ref, degp_ref, b2_ref, o_ref):
    dinv = _dinv_of(degp_ref)
    a = acc_ref[0] + acc_ref[1] - y2_ref[...]
    o_ref[...] = a * dinv[:, None] + b2_ref[0][None, :]


def _mm3(acc2, y2, degp, b2):
    return pl.pallas_call(
        _mm3_body,
        grid=(_GRID,),
        in_specs=[
            pl.BlockSpec((2, _TILE, OUT_CH), lambda i: (0, i, 0)),
            pl.BlockSpec((_TILE, OUT_CH), lambda i: (i, 0)),
            pl.BlockSpec((2, _TILE, 1), lambda i: (0, i, 0)),
            pl.BlockSpec((1, OUT_CH), lambda i: (0, 0)),
        ],
        out_specs=pl.BlockSpec((_TILE, OUT_CH), lambda i: (i, 0)),
        out_shape=jax.ShapeDtypeStruct((N, OUT_CH), jnp.float32),
    )(acc2, y2, degp, b2)


def kernel(x, edge_index, W1, b1, W2, b2):
    e = edge_index.astype(jnp.int32)
    src, dst = e[0], e[1]
    padvals = N + (jnp.arange(PAD, dtype=jnp.int32) % (NP - N))
    pad_blk = jnp.broadcast_to(padvals, (TECS, PAD))
    dst_c = jnp.concatenate([dst.reshape(TECS, EPT), pad_blk], 1).reshape(TECS, CH, K)

    padv_e = N + (jnp.arange(PADE, dtype=jnp.int32) % (NP - N))
    pad_e = jnp.broadcast_to(padv_e, (TECS, PADE))
    src_e = jnp.concatenate([src.reshape(TECS, EPT), pad_e], 1).reshape(
        TECS, NGE, GE, KE)
    dst_e = jnp.concatenate([dst.reshape(TECS, EPT), pad_e], 1).reshape(
        TECS, NGE, GE, KE)
    src_c2 = jnp.stack([src_e, src_e + NP])
    dst_c2 = jnp.stack([dst_e, dst_e])

    padv_e2 = N + (jnp.arange(PADE2, dtype=jnp.int32) % (NP - N))
    pad_e2 = jnp.broadcast_to(padv_e2, (2, TECS, PADE2))
    src_l2 = jnp.concatenate(
        [src.reshape(2, TECS, EPT2), pad_e2], 2).reshape(
            2, TECS, NGE2, GE2, KE)
    dst_l2 = jnp.concatenate(
        [dst.reshape(2, TECS, EPT2), pad_e2], 2).reshape(
            2, TECS, NGE2, GE2, KE)

    _deg_kernel, _edge_l1, _edge_l2 = _sc_kernels()
    degp = _deg_kernel(dst_c).reshape(2, NP, 1)
    y1 = _mm1(x, W1, degp)
    acc1 = _edge_l1(y1.reshape(2 * NP, 128), src_c2, dst_c2)
    y2 = _mm2(acc1.reshape(2, NP, 128), degp, b1.reshape(1, HID), W2)
    acc2 = _edge_l2(y2, src_l2, dst_l2)
    return _mm3(acc2.reshape(2, NP, 128), y2, degp, b2.reshape(1, OUT_CH))


# speedup vs baseline: 1.0330x; 1.0330x over previous
import functools

import jax
import jax.numpy as jnp
from jax import lax
from jax.experimental import pallas as pl
from jax.experimental.pallas import tpu as pltpu
from jax.experimental.pallas import tpu_sc as plsc

N = 10000
NP = 10240
E = 320000
IN_CH = 128
HID = 256
OUT_CH = 128
TECS = 16
K = 128
EPT = E // TECS
CH = 160
PAD = CH * K - EPT
G = 32
NG = CH // G
EPT2 = E // 32
KE = 96
CHE = 210
PADE = CHE * KE - EPT
GE = 42
NGE = CHE // GE
CHE2 = 105
PADE2 = CHE2 * KE - EPT2
GE2 = 21
NGE2 = CHE2 // GE2
RP = NP // TECS

def _mesh():
    return plsc.VectorSubcoreMesh(core_axis_name="c", subcore_axis_name="s",
                                  num_cores=2, num_subcores=TECS)


def _deg_body(dst_hbm, out_hbm, dst_v, ones_v, acc):
    c = lax.axis_index("c")
    s = lax.axis_index("s")

    def fill(i, carry):
        ones_v[pl.ds(i * 16, 16)] = jnp.full((16,), 1.0, jnp.float32)
        return carry

    lax.fori_loop(0, NP // 16, fill, 0)
    pltpu.sync_copy(dst_hbm.at[s], dst_v)

    @pl.when(s == 0)
    def _():
        pltpu.sync_copy(ones_v, acc)

    plsc.subcore_barrier()

    half = CH // 2
    def body(j, carry):
        pltpu.sync_copy(ones_v.at[pl.ds(0, K)], acc.at[dst_v.at[j]], add=True)
        return carry

    lax.fori_loop(half * c, half * (c + 1), body, 0)
    plsc.subcore_barrier()
    pltpu.sync_copy(acc.at[pl.ds(s * RP, RP)], out_hbm.at[c, pl.ds(s * RP, RP)])


def _make_edge_body(colsplit, g_, ng_):
    def body_fn(ys_hbm, src_hbm, dst_hbm, out_hbm,
                src_v, dst_v, rows0, rows1, rows2, acc, sem0, sem1, sem2):
        c = lax.axis_index("c")
        s = lax.axis_index("s")
        rows = (rows0, rows1, rows2)
        sems = (sem0, sem1, sem2)
        init = (c * NP if colsplit else 0) + s * RP
        pltpu.sync_copy(ys_hbm.at[pl.ds(init, RP)], acc.at[pl.ds(s * RP, RP)])
        plsc.subcore_barrier()

        def group(g, carry):
            pltpu.sync_copy(src_hbm.at[c, s, g], src_v)
            pltpu.sync_copy(dst_hbm.at[c, s, g], dst_v)
            pltpu.async_copy(ys_hbm.at[src_v.at[0]], rows0, sem0)
            pltpu.async_copy(ys_hbm.at[src_v.at[1]], rows1, sem1)

            def tri(i, carry2):
                j = 3 * i
                for t in range(3):
                    b, bn = rows[t], rows[(t + 2) % 3]
                    sm, smn = sems[t], sems[(t + 2) % 3]
                    pltpu.make_async_copy(
                        ys_hbm.at[src_v.at[j + t]], b, sm).wait()

                    @pl.when(j + t + 2 < g_)
                    def _():
                        pltpu.async_copy(
                            ys_hbm.at[src_v.at[j + t + 2]], bn, smn)

                    pltpu.sync_copy(b, acc.at[dst_v.at[j + t]], add=True)
                return carry2

            return lax.fori_loop(0, g_ // 3, tri, carry)

        lax.fori_loop(0, ng_, group, 0)
        plsc.subcore_barrier()
        pltpu.sync_copy(acc.at[pl.ds(s * RP, RP)],
                        out_hbm.at[pl.ds(c * NP + s * RP, RP)])

    return body_fn


@functools.cache
def _sc_kernels():
    deg = pl.kernel(
        _deg_body,
        out_type=jax.ShapeDtypeStruct((2, NP), jnp.float32),
        mesh=_mesh(),
        scratch_types=[
            pltpu.VMEM((CH, K), jnp.int32),
            pltpu.VMEM((NP,), jnp.float32),
            pltpu.VMEM_SHARED((NP,), jnp.float32),
        ],
    )
    def edge(colsplit, g, ng, dt):
        return pl.kernel(
            _make_edge_body(colsplit, g, ng),
            out_type=jax.ShapeDtypeStruct((2 * NP, 128), dt),
            mesh=_mesh(),
            scratch_types=[
                pltpu.VMEM((g, KE), jnp.int32),
                pltpu.VMEM((g, KE), jnp.int32),
                pltpu.VMEM((KE, 128), dt),
                pltpu.VMEM((KE, 128), dt),
                pltpu.VMEM((KE, 128), dt),
                pltpu.VMEM_SHARED((NP, 128), dt),
                pltpu.SemaphoreType.DMA,
                pltpu.SemaphoreType.DMA,
                pltpu.SemaphoreType.DMA,
            ],
        )
    return (deg, edge(True, GE, NGE, jnp.float32),
            edge(False, GE2, NGE2, jnp.float32))


_TILE = 512
_GRID = NP // _TILE


def _dinv_of(degp_ref):
    d = degp_ref[0] + degp_ref[1] - 1.0
    return lax.rsqrt(jnp.maximum(d, 1.0))


def _mm1_body(x_ref, w_ref, degp_ref, o_ref):
    dinv = _dinv_of(degp_ref)
    y = jnp.dot(x_ref[...], w_ref[...],
                preferred_element_type=jnp.float32) * dinv[:, None]
    o_ref[0] = y[:, :128]
    o_ref[1] = y[:, 128:]


def _mm1(xp, W1, degp):
    return pl.pallas_call(
        _mm1_body,
        grid=(_GRID,),
        in_specs=[
            pl.BlockSpec((_TILE, IN_CH), lambda i: (i, 0)),
            pl.BlockSpec((IN_CH, HID), lambda i: (0, 0)),
            pl.BlockSpec((2, _TILE), lambda i: (0, i)),
        ],
        out_specs=pl.BlockSpec((2, _TILE, 128), lambda i: (0, i, 0)),
        out_shape=jax.ShapeDtypeStruct((2, NP, 128), jnp.float32),
    )(xp, W1, degp)


def _mm2_body(acc_ref, degp_ref, b1_ref, w_ref, o_ref):
    dinv = _dinv_of(degp_ref)
    a = jnp.concatenate([acc_ref[0], acc_ref[1]], axis=1)
    h = jax.nn.relu(a * dinv[:, None] + b1_ref[0][None, :])
    y2 = jnp.dot(h, w_ref[...],
                 preferred_element_type=jnp.float32) * dinv[:, None]
    o_ref[...] = y2


def _mm2(acc1, degp, b1, W2):
    return pl.pallas_call(
        _mm2_body,
        grid=(_GRID,),
        in_specs=[
            pl.BlockSpec((2, _TILE, 128), lambda i: (0, i, 0)),
            pl.BlockSpec((2, _TILE), lambda i: (0, i)),
            pl.BlockSpec((1, HID), lambda i: (0, 0)),
            pl.BlockSpec((HID, OUT_CH), lambda i: (0, 0)),
        ],
        out_specs=pl.BlockSpec((_TILE, OUT_CH), lambda i: (i, 0)),
        out_shape=jax.ShapeDtypeStruct((NP, OUT_CH), jnp.float32),
    )(acc1, degp, b1, W2)


def _mm3_body(acc_ref, y2_ref, degp_ref, b2_ref, o_ref):
    dinv = _dinv_of(degp_ref)
    a = acc_ref[0] + acc_ref[1] - y2_ref[...]
    o_ref[...] = a * dinv[:, None] + b2_ref[0][None, :]


def _mm3(acc2, y2, degp, b2):
    return pl.pallas_call(
        _mm3_body,
        grid=(_GRID,),
        in_specs=[
            pl.BlockSpec((2, _TILE, OUT_CH), lambda i: (0, i, 0)),
            pl.BlockSpec((_TILE, OUT_CH), lambda i: (i, 0)),
            pl.BlockSpec((2, _TILE), lambda i: (0, i)),
            pl.BlockSpec((1, OUT_CH), lambda i: (0, 0)),
        ],
        out_specs=pl.BlockSpec((_TILE, OUT_CH), lambda i: (i, 0)),
        out_shape=jax.ShapeDtypeStruct((NP, OUT_CH), jnp.float32),
    )(acc2, y2, degp, b2)


def kernel(x, edge_index, W1, b1, W2, b2):
    e = edge_index.astype(jnp.int32)
    src, dst = e[0], e[1]
    padvals = N + (jnp.arange(PAD, dtype=jnp.int32) % (NP - N))
    pad_blk = jnp.broadcast_to(padvals, (TECS, PAD))
    dst_c = jnp.concatenate([dst.reshape(TECS, EPT), pad_blk], 1).reshape(TECS, CH, K)

    padv_e = N + (jnp.arange(PADE, dtype=jnp.int32) % (NP - N))
    pad_e = jnp.broadcast_to(padv_e, (TECS, PADE))
    src_e = jnp.concatenate([src.reshape(TECS, EPT), pad_e], 1).reshape(
        TECS, NGE, GE, KE)
    dst_e = jnp.concatenate([dst.reshape(TECS, EPT), pad_e], 1).reshape(
        TECS, NGE, GE, KE)
    src_c2 = jnp.stack([src_e, src_e + NP])
    dst_c2 = jnp.stack([dst_e, dst_e])

    padv_e2 = N + (jnp.arange(PADE2, dtype=jnp.int32) % (NP - N))
    pad_e2 = jnp.broadcast_to(padv_e2, (2, TECS, PADE2))
    src_l2 = jnp.concatenate(
        [src.reshape(2, TECS, EPT2), pad_e2], 2).reshape(
            2, TECS, NGE2, GE2, KE)
    dst_l2 = jnp.concatenate(
        [dst.reshape(2, TECS, EPT2), pad_e2], 2).reshape(
            2, TECS, NGE2, GE2, KE)

    xp = jnp.concatenate(
        [x, jnp.zeros((NP - N, IN_CH), jnp.float32)], axis=0)

    _deg_kernel, _edge_l1, _edge_l2 = _sc_kernels()
    degp = _deg_kernel(dst_c)
    y1 = _mm1(xp, W1, degp)
    acc1 = _edge_l1(y1.reshape(2 * NP, 128), src_c2, dst_c2)
    y2 = _mm2(acc1.reshape(2, NP, 128), degp, b1.reshape(1, HID), W2)
    acc2 = _edge_l2(y2, src_l2, dst_l2)
    out = _mm3(acc2.reshape(2, NP, 128), y2, degp, b2.reshape(1, OUT_CH))
    return out[:N]


# speedup vs baseline: 1.0731x; 1.0389x over previous
import functools

import jax
import jax.numpy as jnp
from jax import lax
from jax.experimental import pallas as pl
from jax.experimental.pallas import tpu as pltpu
from jax.experimental.pallas import tpu_sc as plsc

N = 10000
NP = 10240
E = 320000
IN_CH = 128
HID = 256
OUT_CH = 128
TECS = 16
K = 128
EPT = E // TECS
CH = 160
PAD = CH * K - EPT
G = 32
NG = CH // G
EPT2 = E // 32
KE = 96
CHE = 210
PADE = CHE * KE - EPT
GE = 42
NGE = CHE // GE
CHE2 = 105
PADE2 = CHE2 * KE - EPT2
GE2 = 21
NGE2 = CHE2 // GE2
RP = NP // TECS

def _mesh():
    return plsc.VectorSubcoreMesh(core_axis_name="c", subcore_axis_name="s",
                                  num_cores=2, num_subcores=TECS)


def _deg_body(dst_hbm, out_hbm, dst_v, ones_v, acc):
    c = lax.axis_index("c")
    s = lax.axis_index("s")

    def fill(i, carry):
        ones_v[pl.ds(i * 16, 16)] = jnp.full((16,), 1.0, jnp.float32)
        return carry

    lax.fori_loop(0, NP // 16, fill, 0)
    pltpu.sync_copy(dst_hbm.at[s], dst_v)

    @pl.when(s == 0)
    def _():
        pltpu.sync_copy(ones_v, acc)

    plsc.subcore_barrier()

    half = CH // 2
    def body(j, carry):
        pltpu.sync_copy(ones_v.at[pl.ds(0, K)], acc.at[dst_v.at[j]], add=True)
        return carry

    lax.fori_loop(half * c, half * (c + 1), body, 0)
    plsc.subcore_barrier()
    pltpu.sync_copy(acc.at[pl.ds(s * RP, RP)], out_hbm.at[c, pl.ds(s * RP, RP)])


def _make_edge_body(colsplit, g_, ng_):
    def body_fn(ys_hbm, src_hbm, dst_hbm, out_hbm,
                src_v, dst_v, rows0, rows1, rows2, acc, sem0, sem1, sem2):
        c = lax.axis_index("c")
        s = lax.axis_index("s")
        rows = (rows0, rows1, rows2)
        sems = (sem0, sem1, sem2)
        init = (c * NP if colsplit else 0) + s * RP
        pltpu.sync_copy(ys_hbm.at[pl.ds(init, RP)], acc.at[pl.ds(s * RP, RP)])
        plsc.subcore_barrier()

        def group(g, carry):
            pltpu.sync_copy(src_hbm.at[c, s, g], src_v)
            pltpu.sync_copy(dst_hbm.at[c, s, g], dst_v)
            pltpu.async_copy(ys_hbm.at[src_v.at[0]], rows0, sem0)
            pltpu.async_copy(ys_hbm.at[src_v.at[1]], rows1, sem1)

            def tri(i, carry2):
                j = 3 * i
                for t in range(3):
                    b, bn = rows[t], rows[(t + 2) % 3]
                    sm, smn = sems[t], sems[(t + 2) % 3]
                    pltpu.make_async_copy(
                        ys_hbm.at[src_v.at[j + t]], b, sm).wait()

                    @pl.when(j + t + 2 < g_)
                    def _():
                        pltpu.async_copy(
                            ys_hbm.at[src_v.at[j + t + 2]], bn, smn)

                    pltpu.sync_copy(b, acc.at[dst_v.at[j + t]], add=True)
                return carry2

            return lax.fori_loop(0, g_ // 3, tri, carry)

        lax.fori_loop(0, ng_, group, 0)
        plsc.subcore_barrier()
        pltpu.sync_copy(acc.at[pl.ds(s * RP, RP)],
                        out_hbm.at[pl.ds(c * NP + s * RP, RP)])

    return body_fn


@functools.cache
def _sc_kernels():
    deg = pl.kernel(
        _deg_body,
        out_type=jax.ShapeDtypeStruct((2, NP), jnp.float32),
        mesh=_mesh(),
        scratch_types=[
            pltpu.VMEM((CH, K), jnp.int32),
            pltpu.VMEM((NP,), jnp.float32),
            pltpu.VMEM_SHARED((NP,), jnp.float32),
        ],
    )
    def edge(colsplit, g, ng, dt):
        return pl.kernel(
            _make_edge_body(colsplit, g, ng),
            out_type=jax.ShapeDtypeStruct((2 * NP, 128), dt),
            mesh=_mesh(),
            scratch_types=[
                pltpu.VMEM((g, KE), jnp.int32),
                pltpu.VMEM((g, KE), jnp.int32),
                pltpu.VMEM((KE, 128), dt),
                pltpu.VMEM((KE, 128), dt),
                pltpu.VMEM((KE, 128), dt),
                pltpu.VMEM_SHARED((NP, 128), dt),
                pltpu.SemaphoreType.DMA,
                pltpu.SemaphoreType.DMA,
                pltpu.SemaphoreType.DMA,
            ],
        )
    return (deg, edge(True, GE, NGE, jnp.float32),
            edge(False, GE2, NGE2, jnp.float32))


_TILE = 1024
_GRID = NP // _TILE


def _dinv_of(degp_ref):
    d = degp_ref[0] + degp_ref[1] - 1.0
    return lax.rsqrt(jnp.maximum(d, 1.0))


def _mm1_body(x_ref, w_ref, degp_ref, o_ref):
    dinv = _dinv_of(degp_ref)
    y = jnp.dot(x_ref[...], w_ref[...],
                preferred_element_type=jnp.float32) * dinv[:, None]
    o_ref[0] = y[:, :128]
    o_ref[1] = y[:, 128:]


def _mm1(xp, W1, degp):
    return pl.pallas_call(
        _mm1_body,
        grid=(_GRID,),
        in_specs=[
            pl.BlockSpec((_TILE, IN_CH), lambda i: (i, 0)),
            pl.BlockSpec((IN_CH, HID), lambda i: (0, 0)),
            pl.BlockSpec((2, _TILE), lambda i: (0, i)),
        ],
        out_specs=pl.BlockSpec((2, _TILE, 128), lambda i: (0, i, 0)),
        out_shape=jax.ShapeDtypeStruct((2, NP, 128), jnp.float32),
    )(xp, W1, degp)


def _mm2_body(acc_ref, degp_ref, b1_ref, w_ref, o_ref):
    dinv = _dinv_of(degp_ref)
    a = jnp.concatenate([acc_ref[0], acc_ref[1]], axis=1)
    h = jax.nn.relu(a * dinv[:, None] + b1_ref[0][None, :])
    y2 = jnp.dot(h, w_ref[...],
                 preferred_element_type=jnp.float32) * dinv[:, None]
    o_ref[...] = y2


def _mm2(acc1, degp, b1, W2):
    return pl.pallas_call(
        _mm2_body,
        grid=(_GRID,),
        in_specs=[
            pl.BlockSpec((2, _TILE, 128), lambda i: (0, i, 0)),
            pl.BlockSpec((2, _TILE), lambda i: (0, i)),
            pl.BlockSpec((1, HID), lambda i: (0, 0)),
            pl.BlockSpec((HID, OUT_CH), lambda i: (0, 0)),
        ],
        out_specs=pl.BlockSpec((_TILE, OUT_CH), lambda i: (i, 0)),
        out_shape=jax.ShapeDtypeStruct((NP, OUT_CH), jnp.float32),
    )(acc1, degp, b1, W2)


def _mm3_body(acc_ref, y2_ref, degp_ref, b2_ref, o_ref):
    dinv = _dinv_of(degp_ref)
    a = acc_ref[0] + acc_ref[1] - y2_ref[...]
    o_ref[...] = a * dinv[:, None] + b2_ref[0][None, :]


def _mm3(acc2, y2, degp, b2):
    return pl.pallas_call(
        _mm3_body,
        grid=(_GRID,),
        in_specs=[
            pl.BlockSpec((2, _TILE, OUT_CH), lambda i: (0, i, 0)),
            pl.BlockSpec((_TILE, OUT_CH), lambda i: (i, 0)),
            pl.BlockSpec((2, _TILE), lambda i: (0, i)),
            pl.BlockSpec((1, OUT_CH), lambda i: (0, 0)),
        ],
        out_specs=pl.BlockSpec((_TILE, OUT_CH), lambda i: (i, 0)),
        out_shape=jax.ShapeDtypeStruct((NP, OUT_CH), jnp.float32),
    )(acc2, y2, degp, b2)


def kernel(x, edge_index, W1, b1, W2, b2):
    e = edge_index.astype(jnp.int32)
    src, dst = e[0], e[1]
    padvals = N + (jnp.arange(PAD, dtype=jnp.int32) % (NP - N))
    pad_blk = jnp.broadcast_to(padvals, (TECS, PAD))
    dst_c = jnp.concatenate([dst.reshape(TECS, EPT), pad_blk], 1).reshape(TECS, CH, K)

    padv_e = N + (jnp.arange(PADE, dtype=jnp.int32) % (NP - N))
    pad_e = jnp.broadcast_to(padv_e, (TECS, PADE))
    src_e = jnp.concatenate([src.reshape(TECS, EPT), pad_e], 1).reshape(
        TECS, NGE, GE, KE)
    dst_e = jnp.concatenate([dst.reshape(TECS, EPT), pad_e], 1).reshape(
        TECS, NGE, GE, KE)
    src_c2 = jnp.stack([src_e, src_e + NP])
    dst_c2 = jnp.stack([dst_e, dst_e])

    padv_e2 = N + (jnp.arange(PADE2, dtype=jnp.int32) % (NP - N))
    pad_e2 = jnp.broadcast_to(padv_e2, (2, TECS, PADE2))
    src_l2 = jnp.concatenate(
        [src.reshape(2, TECS, EPT2), pad_e2], 2).reshape(
            2, TECS, NGE2, GE2, KE)
    dst_l2 = jnp.concatenate(
        [dst.reshape(2, TECS, EPT2), pad_e2], 2).reshape(
            2, TECS, NGE2, GE2, KE)

    xp = jnp.concatenate(
        [x, jnp.zeros((NP - N, IN_CH), jnp.float32)], axis=0)

    _deg_kernel, _edge_l1, _edge_l2 = _sc_kernels()
    degp = _deg_kernel(dst_c)
    y1 = _mm1(xp, W1, degp)
    acc1 = _edge_l1(y1.reshape(2 * NP, 128), src_c2, dst_c2)
    y2 = _mm2(acc1.reshape(2, NP, 128), degp, b1.reshape(1, HID), W2)
    acc2 = _edge_l2(y2, src_l2, dst_l2)
    out = _mm3(acc2.reshape(2, NP, 128), y2, degp, b2.reshape(1, OUT_CH))
    return out[:N]


# speedup vs baseline: 1.0942x; 1.0197x over previous
import functools

import jax
import jax.numpy as jnp
from jax import lax
from jax.experimental import pallas as pl
from jax.experimental.pallas import tpu as pltpu
from jax.experimental.pallas import tpu_sc as plsc

N = 10000
NP = 10240
E = 320000
IN_CH = 128
HID = 256
OUT_CH = 128
TECS = 16
K = 128
EPT = E // TECS
CH = 160
PAD = CH * K - EPT
G = 32
NG = CH // G
EPT2 = E // 32
KE = 96
CHE = 210
PADE = CHE * KE - EPT
GE = 42
NGE = CHE // GE
CHE2 = 105
PADE2 = CHE2 * KE - EPT2
GE2 = 21
NGE2 = CHE2 // GE2
RP = NP // TECS

def _mesh():
    return plsc.VectorSubcoreMesh(core_axis_name="c", subcore_axis_name="s",
                                  num_cores=2, num_subcores=TECS)


def _deg_body(dst_hbm, out_hbm, dst_v, ones_v, acc):
    c = lax.axis_index("c")
    s = lax.axis_index("s")

    def fill(i, carry):
        ones_v[pl.ds(i * 16, 16)] = jnp.full((16,), 1.0, jnp.float32)
        return carry

    lax.fori_loop(0, NP // 16, fill, 0)
    pltpu.sync_copy(dst_hbm.at[s], dst_v)

    @pl.when(s == 0)
    def _():
        pltpu.sync_copy(ones_v, acc)

    plsc.subcore_barrier()

    half = CH // 2
    def body(j, carry):
        pltpu.sync_copy(ones_v.at[pl.ds(0, K)], acc.at[dst_v.at[j]], add=True)
        return carry

    lax.fori_loop(half * c, half * (c + 1), body, 0)
    plsc.subcore_barrier()
    pltpu.sync_copy(acc.at[pl.ds(s * RP, RP)], out_hbm.at[c, pl.ds(s * RP, RP)])


def _make_edge_body(colsplit, g_, ng_):
    def body_fn(ys_hbm, src_hbm, dst_hbm, out_hbm,
                src_v, dst_v, rows0, rows1, rows2, acc, sem0, sem1, sem2):
        c = lax.axis_index("c")
        s = lax.axis_index("s")
        rows = (rows0, rows1, rows2)
        sems = (sem0, sem1, sem2)
        init = (c * NP if colsplit else 0) + s * RP
        pltpu.sync_copy(ys_hbm.at[pl.ds(init, RP)], acc.at[pl.ds(s * RP, RP)])
        plsc.subcore_barrier()

        def group(g, carry):
            pltpu.sync_copy(src_hbm.at[c, s, g], src_v)
            pltpu.sync_copy(dst_hbm.at[c, s, g], dst_v)
            pltpu.async_copy(ys_hbm.at[src_v.at[0]], rows0, sem0)
            pltpu.async_copy(ys_hbm.at[src_v.at[1]], rows1, sem1)

            def tri(i, carry2):
                j = 3 * i
                for t in range(3):
                    b, bn = rows[t], rows[(t + 2) % 3]
                    sm, smn = sems[t], sems[(t + 2) % 3]
                    pltpu.make_async_copy(
                        ys_hbm.at[src_v.at[j + t]], b, sm).wait()

                    @pl.when(j + t + 2 < g_)
                    def _():
                        pltpu.async_copy(
                            ys_hbm.at[src_v.at[j + t + 2]], bn, smn)

                    pltpu.sync_copy(b, acc.at[dst_v.at[j + t]], add=True)
                return carry2

            return lax.fori_loop(0, g_ // 3, tri, carry)

        lax.fori_loop(0, ng_, group, 0)
        plsc.subcore_barrier()
        pltpu.sync_copy(acc.at[pl.ds(s * RP, RP)],
                        out_hbm.at[pl.ds(c * NP + s * RP, RP)])

    return body_fn


@functools.cache
def _sc_kernels():
    deg = pl.kernel(
        _deg_body,
        out_type=jax.ShapeDtypeStruct((2, NP), jnp.float32),
        mesh=_mesh(),
        scratch_types=[
            pltpu.VMEM((CH, K), jnp.int32),
            pltpu.VMEM((NP,), jnp.float32),
            pltpu.VMEM_SHARED((NP,), jnp.float32),
        ],
    )
    def edge(colsplit, g, ng, dt):
        return pl.kernel(
            _make_edge_body(colsplit, g, ng),
            out_type=jax.ShapeDtypeStruct((2 * NP, 128), dt),
            mesh=_mesh(),
            scratch_types=[
                pltpu.VMEM((g, KE), jnp.int32),
                pltpu.VMEM((g, KE), jnp.int32),
                pltpu.VMEM((KE, 128), dt),
                pltpu.VMEM((KE, 128), dt),
                pltpu.VMEM((KE, 128), dt),
                pltpu.VMEM_SHARED((NP, 128), dt),
                pltpu.SemaphoreType.DMA,
                pltpu.SemaphoreType.DMA,
                pltpu.SemaphoreType.DMA,
            ],
        )
    return (deg, edge(True, GE, NGE, jnp.float32),
            edge(False, GE2, NGE2, jnp.float32))


_TILE = 2048
_GRID = NP // _TILE


def _dinv_of(degp_ref):
    d = degp_ref[0] + degp_ref[1] - 1.0
    return lax.rsqrt(jnp.maximum(d, 1.0))


def _mm1_body(x_ref, w_ref, degp_ref, o_ref):
    dinv = _dinv_of(degp_ref)
    y = jnp.dot(x_ref[...], w_ref[...],
                preferred_element_type=jnp.float32) * dinv[:, None]
    o_ref[0] = y[:, :128]
    o_ref[1] = y[:, 128:]


def _mm1(xp, W1, degp):
    return pl.pallas_call(
        _mm1_body,
        grid=(_GRID,),
        in_specs=[
            pl.BlockSpec((_TILE, IN_CH), lambda i: (i, 0)),
            pl.BlockSpec((IN_CH, HID), lambda i: (0, 0)),
            pl.BlockSpec((2, _TILE), lambda i: (0, i)),
        ],
        out_specs=pl.BlockSpec((2, _TILE, 128), lambda i: (0, i, 0)),
        out_shape=jax.ShapeDtypeStruct((2, NP, 128), jnp.float32),
    )(xp, W1, degp)


def _mm2_body(acc_ref, degp_ref, b1_ref, w_ref, o_ref):
    dinv = _dinv_of(degp_ref)
    a = jnp.concatenate([acc_ref[0], acc_ref[1]], axis=1)
    h = jax.nn.relu(a * dinv[:, None] + b1_ref[0][None, :])
    y2 = jnp.dot(h, w_ref[...],
                 preferred_element_type=jnp.float32) * dinv[:, None]
    o_ref[...] = y2


def _mm2(acc1, degp, b1, W2):
    return pl.pallas_call(
        _mm2_body,
        grid=(_GRID,),
        in_specs=[
            pl.BlockSpec((2, _TILE, 128), lambda i: (0, i, 0)),
            pl.BlockSpec((2, _TILE), lambda i: (0, i)),
            pl.BlockSpec((1, HID), lambda i: (0, 0)),
            pl.BlockSpec((HID, OUT_CH), lambda i: (0, 0)),
        ],
        out_specs=pl.BlockSpec((_TILE, OUT_CH), lambda i: (i, 0)),
        out_shape=jax.ShapeDtypeStruct((NP, OUT_CH), jnp.float32),
    )(acc1, degp, b1, W2)


def _mm3_body(acc_ref, y2_ref, degp_ref, b2_ref, o_ref):
    dinv = _dinv_of(degp_ref)
    a = acc_ref[0] + acc_ref[1] - y2_ref[...]
    o_ref[...] = a * dinv[:, None] + b2_ref[0][None, :]


def _mm3(acc2, y2, degp, b2):
    return pl.pallas_call(
        _mm3_body,
        grid=(_GRID,),
        in_specs=[
            pl.BlockSpec((2, _TILE, OUT_CH), lambda i: (0, i, 0)),
            pl.BlockSpec((_TILE, OUT_CH), lambda i: (i, 0)),
            pl.BlockSpec((2, _TILE), lambda i: (0, i)),
            pl.BlockSpec((1, OUT_CH), lambda i: (0, 0)),
        ],
        out_specs=pl.BlockSpec((_TILE, OUT_CH), lambda i: (i, 0)),
        out_shape=jax.ShapeDtypeStruct((NP, OUT_CH), jnp.float32),
    )(acc2, y2, degp, b2)


def kernel(x, edge_index, W1, b1, W2, b2):
    e = edge_index.astype(jnp.int32)
    src, dst = e[0], e[1]
    padvals = N + (jnp.arange(PAD, dtype=jnp.int32) % (NP - N))
    pad_blk = jnp.broadcast_to(padvals, (TECS, PAD))
    dst_c = jnp.concatenate([dst.reshape(TECS, EPT), pad_blk], 1).reshape(TECS, CH, K)

    padv_e = N + (jnp.arange(PADE, dtype=jnp.int32) % (NP - N))
    pad_e = jnp.broadcast_to(padv_e, (TECS, PADE))
    src_e = jnp.concatenate([src.reshape(TECS, EPT), pad_e], 1).reshape(
        TECS, NGE, GE, KE)
    dst_e = jnp.concatenate([dst.reshape(TECS, EPT), pad_e], 1).reshape(
        TECS, NGE, GE, KE)
    src_c2 = jnp.stack([src_e, src_e + NP])
    dst_c2 = jnp.stack([dst_e, dst_e])

    padv_e2 = N + (jnp.arange(PADE2, dtype=jnp.int32) % (NP - N))
    pad_e2 = jnp.broadcast_to(padv_e2, (2, TECS, PADE2))
    src_l2 = jnp.concatenate(
        [src.reshape(2, TECS, EPT2), pad_e2], 2).reshape(
            2, TECS, NGE2, GE2, KE)
    dst_l2 = jnp.concatenate(
        [dst.reshape(2, TECS, EPT2), pad_e2], 2).reshape(
            2, TECS, NGE2, GE2, KE)

    xp = jnp.concatenate(
        [x, jnp.zeros((NP - N, IN_CH), jnp.float32)], axis=0)

    _deg_kernel, _edge_l1, _edge_l2 = _sc_kernels()
    degp = _deg_kernel(dst_c)
    y1 = _mm1(xp, W1, degp)
    acc1 = _edge_l1(y1.reshape(2 * NP, 128), src_c2, dst_c2)
    y2 = _mm2(acc1.reshape(2, NP, 128), degp, b1.reshape(1, HID), W2)
    acc2 = _edge_l2(y2, src_l2, dst_l2)
    out = _mm3(acc2.reshape(2, NP, 128), y2, degp, b2.reshape(1, OUT_CH))
    return out[:N]


# speedup vs baseline: 1.1078x; 1.0124x over previous
import functools

import jax
import jax.numpy as jnp
from jax import lax
from jax.experimental import pallas as pl
from jax.experimental.pallas import tpu as pltpu
from jax.experimental.pallas import tpu_sc as plsc

N = 10000
NP = 10240
E = 320000
IN_CH = 128
HID = 256
OUT_CH = 128
TECS = 16
K = 128
EPT = E // TECS
CH = 160
PAD = CH * K - EPT
G = 32
NG = CH // G
EPT2 = E // 32
KE = 96
CHE = 210
PADE = CHE * KE - EPT
GE = 42
NGE = CHE // GE
CHE2 = 105
PADE2 = CHE2 * KE - EPT2
GE2 = 21
NGE2 = CHE2 // GE2
RP = NP // TECS

def _mesh():
    return plsc.VectorSubcoreMesh(core_axis_name="c", subcore_axis_name="s",
                                  num_cores=2, num_subcores=TECS)


def _deg_body(dst_hbm, out_hbm, dst_v, ones_v, acc):
    c = lax.axis_index("c")
    s = lax.axis_index("s")

    def fill(i, carry):
        ones_v[pl.ds(i * 16, 16)] = jnp.full((16,), 1.0, jnp.float32)
        return carry

    lax.fori_loop(0, NP // 16, fill, 0)
    pltpu.sync_copy(dst_hbm.at[s], dst_v)

    @pl.when(s == 0)
    def _():
        pltpu.sync_copy(ones_v, acc)

    plsc.subcore_barrier()

    half = CH // 2
    def body(j, carry):
        pltpu.sync_copy(ones_v.at[pl.ds(0, K)], acc.at[dst_v.at[j]], add=True)
        return carry

    lax.fori_loop(half * c, half * (c + 1), body, 0)
    plsc.subcore_barrier()
    pltpu.sync_copy(acc.at[pl.ds(s * RP, RP)], out_hbm.at[c, pl.ds(s * RP, RP)])


def _make_edge_body(colsplit, g_, ng_):
    def body_fn(ys_hbm, src_hbm, dst_hbm, out_hbm,
                src_v, dst_v, rows0, rows1, rows2, acc, sem0, sem1, sem2):
        c = lax.axis_index("c")
        s = lax.axis_index("s")
        rows = (rows0, rows1, rows2)
        sems = (sem0, sem1, sem2)
        init = (c * NP if colsplit else 0) + s * RP
        pltpu.sync_copy(ys_hbm.at[pl.ds(init, RP)], acc.at[pl.ds(s * RP, RP)])
        plsc.subcore_barrier()

        def group(g, carry):
            pltpu.sync_copy(src_hbm.at[c, s, g], src_v)
            pltpu.sync_copy(dst_hbm.at[c, s, g], dst_v)
            pltpu.async_copy(ys_hbm.at[src_v.at[0]], rows0, sem0)
            pltpu.async_copy(ys_hbm.at[src_v.at[1]], rows1, sem1)

            def tri(i, carry2):
                j = 3 * i
                for t in range(3):
                    b, bn = rows[t], rows[(t + 2) % 3]
                    sm, smn = sems[t], sems[(t + 2) % 3]
                    pltpu.make_async_copy(
                        ys_hbm.at[src_v.at[j + t]], b, sm).wait()

                    @pl.when(j + t + 2 < g_)
                    def _():
                        pltpu.async_copy(
                            ys_hbm.at[src_v.at[j + t + 2]], bn, smn)

                    pltpu.sync_copy(b, acc.at[dst_v.at[j + t]], add=True)
                return carry2

            return lax.fori_loop(0, g_ // 3, tri, carry)

        lax.fori_loop(0, ng_, group, 0)
        plsc.subcore_barrier()
        pltpu.sync_copy(acc.at[pl.ds(s * RP, RP)],
                        out_hbm.at[pl.ds(c * NP + s * RP, RP)])

    return body_fn


@functools.cache
def _sc_kernels():
    deg = pl.kernel(
        _deg_body,
        out_type=jax.ShapeDtypeStruct((2, NP), jnp.float32),
        mesh=_mesh(),
        scratch_types=[
            pltpu.VMEM((CH, K), jnp.int32),
            pltpu.VMEM((NP,), jnp.float32),
            pltpu.VMEM_SHARED((NP,), jnp.float32),
        ],
    )
    def edge(colsplit, g, ng, dt):
        return pl.kernel(
            _make_edge_body(colsplit, g, ng),
            out_type=jax.ShapeDtypeStruct((2 * NP, 128), dt),
            mesh=_mesh(),
            scratch_types=[
                pltpu.VMEM((g, KE), jnp.int32),
                pltpu.VMEM((g, KE), jnp.int32),
                pltpu.VMEM((KE, 128), dt),
                pltpu.VMEM((KE, 128), dt),
                pltpu.VMEM((KE, 128), dt),
                pltpu.VMEM_SHARED((NP, 128), dt),
                pltpu.SemaphoreType.DMA,
                pltpu.SemaphoreType.DMA,
                pltpu.SemaphoreType.DMA,
            ],
        )
    return (deg, edge(True, GE, NGE, jnp.float32),
            edge(False, GE2, NGE2, jnp.float32))


_TILE = 5120
_GRID = NP // _TILE


def _dinv_of(degp_ref):
    d = degp_ref[0] + degp_ref[1] - 1.0
    return lax.rsqrt(jnp.maximum(d, 1.0))


def _mm1_body(x_ref, w_ref, degp_ref, o_ref):
    dinv = _dinv_of(degp_ref)
    y = jnp.dot(x_ref[...], w_ref[...],
                preferred_element_type=jnp.float32) * dinv[:, None]
    o_ref[0] = y[:, :128]
    o_ref[1] = y[:, 128:]


def _mm1(xp, W1, degp):
    return pl.pallas_call(
        _mm1_body,
        grid=(_GRID,),
        in_specs=[
            pl.BlockSpec((_TILE, IN_CH), lambda i: (i, 0)),
            pl.BlockSpec((IN_CH, HID), lambda i: (0, 0)),
            pl.BlockSpec((2, _TILE), lambda i: (0, i)),
        ],
        out_specs=pl.BlockSpec((2, _TILE, 128), lambda i: (0, i, 0)),
        out_shape=jax.ShapeDtypeStruct((2, NP, 128), jnp.float32),
    )(xp, W1, degp)


def _mm2_body(acc_ref, degp_ref, b1_ref, w_ref, o_ref):
    dinv = _dinv_of(degp_ref)
    a = jnp.concatenate([acc_ref[0], acc_ref[1]], axis=1)
    h = jax.nn.relu(a * dinv[:, None] + b1_ref[0][None, :])
    y2 = jnp.dot(h, w_ref[...],
                 preferred_element_type=jnp.float32) * dinv[:, None]
    o_ref[...] = y2


def _mm2(acc1, degp, b1, W2):
    return pl.pallas_call(
        _mm2_body,
        grid=(_GRID,),
        in_specs=[
            pl.BlockSpec((2, _TILE, 128), lambda i: (0, i, 0)),
            pl.BlockSpec((2, _TILE), lambda i: (0, i)),
            pl.BlockSpec((1, HID), lambda i: (0, 0)),
            pl.BlockSpec((HID, OUT_CH), lambda i: (0, 0)),
        ],
        out_specs=pl.BlockSpec((_TILE, OUT_CH), lambda i: (i, 0)),
        out_shape=jax.ShapeDtypeStruct((NP, OUT_CH), jnp.float32),
    )(acc1, degp, b1, W2)


def _mm3_body(acc_ref, y2_ref, degp_ref, b2_ref, o_ref):
    dinv = _dinv_of(degp_ref)
    a = acc_ref[0] + acc_ref[1] - y2_ref[...]
    o_ref[...] = a * dinv[:, None] + b2_ref[0][None, :]


def _mm3(acc2, y2, degp, b2):
    return pl.pallas_call(
        _mm3_body,
        grid=(_GRID,),
        in_specs=[
            pl.BlockSpec((2, _TILE, OUT_CH), lambda i: (0, i, 0)),
            pl.BlockSpec((_TILE, OUT_CH), lambda i: (i, 0)),
            pl.BlockSpec((2, _TILE), lambda i: (0, i)),
            pl.BlockSpec((1, OUT_CH), lambda i: (0, 0)),
        ],
        out_specs=pl.BlockSpec((_TILE, OUT_CH), lambda i: (i, 0)),
        out_shape=jax.ShapeDtypeStruct((NP, OUT_CH), jnp.float32),
    )(acc2, y2, degp, b2)


def kernel(x, edge_index, W1, b1, W2, b2):
    e = edge_index.astype(jnp.int32)
    src, dst = e[0], e[1]
    padvals = N + (jnp.arange(PAD, dtype=jnp.int32) % (NP - N))
    pad_blk = jnp.broadcast_to(padvals, (TECS, PAD))
    dst_c = jnp.concatenate([dst.reshape(TECS, EPT), pad_blk], 1).reshape(TECS, CH, K)

    padv_e = N + (jnp.arange(PADE, dtype=jnp.int32) % (NP - N))
    pad_e = jnp.broadcast_to(padv_e, (TECS, PADE))
    src_e = jnp.concatenate([src.reshape(TECS, EPT), pad_e], 1).reshape(
        TECS, NGE, GE, KE)
    dst_e = jnp.concatenate([dst.reshape(TECS, EPT), pad_e], 1).reshape(
        TECS, NGE, GE, KE)
    src_c2 = jnp.stack([src_e, src_e + NP])
    dst_c2 = jnp.stack([dst_e, dst_e])

    padv_e2 = N + (jnp.arange(PADE2, dtype=jnp.int32) % (NP - N))
    pad_e2 = jnp.broadcast_to(padv_e2, (2, TECS, PADE2))
    src_l2 = jnp.concatenate(
        [src.reshape(2, TECS, EPT2), pad_e2], 2).reshape(
            2, TECS, NGE2, GE2, KE)
    dst_l2 = jnp.concatenate(
        [dst.reshape(2, TECS, EPT2), pad_e2], 2).reshape(
            2, TECS, NGE2, GE2, KE)

    xp = jnp.concatenate(
        [x, jnp.zeros((NP - N, IN_CH), jnp.float32)], axis=0)

    _deg_kernel, _edge_l1, _edge_l2 = _sc_kernels()
    degp = _deg_kernel(dst_c)
    y1 = _mm1(xp, W1, degp)
    acc1 = _edge_l1(y1.reshape(2 * NP, 128), src_c2, dst_c2)
    y2 = _mm2(acc1.reshape(2, NP, 128), degp, b1.reshape(1, HID), W2)
    acc2 = _edge_l2(y2, src_l2, dst_l2)
    out = _mm3(acc2.reshape(2, NP, 128), y2, degp, b2.reshape(1, OUT_CH))
    return out[:N]
